# Initial kernel scaffold; baseline (speedup 1.0000x reference)
import jax
import jax.numpy as jnp
from jax.experimental import pallas as pl


def kernel(L_x_, L_edge_index_, L_self_modules_convs_modules_0_modules_lin_parameters_weight_, L_self_modules_convs_modules_0_parameters_att_src_, L_self_modules_convs_modules_0_parameters_att_dst_, L_self_modules_convs_modules_0_parameters_bias_, L_self_modules_convs_modules_1_modules_lin_parameters_weight_, L_self_modules_convs_modules_1_parameters_att_src_, L_self_modules_convs_modules_1_parameters_att_dst_, L_self_modules_convs_modules_1_parameters_bias_, L_self_modules_convs_modules_2_modules_lin_parameters_weight_, L_self_modules_convs_modules_2_parameters_att_src_, L_self_modules_convs_modules_2_parameters_att_dst_, L_self_modules_convs_modules_2_parameters_bias_, L_self_modules_convs_modules_3_modules_lin_parameters_weight_, L_self_modules_convs_modules_3_parameters_att_src_, L_self_modules_convs_modules_3_parameters_att_dst_, L_self_modules_convs_modules_3_parameters_bias_, L_self_modules_convs_modules_4_modules_lin_parameters_weight_, L_self_modules_convs_modules_4_parameters_att_src_, L_self_modules_convs_modules_4_parameters_att_dst_, L_self_modules_convs_modules_4_parameters_bias_):
    raise NotImplementedError("write your pallas kernel here")


# speedup vs baseline: 35.1920x; 35.1920x over previous
import jax
import jax.numpy as jnp
from jax import lax
from jax.experimental import pallas as pl

N = 1000
H = 4
D = 64
E = 100
EP = 128
HP = 8
HID = H * D
L = 5


def _dot(a, b):
    return jnp.dot(a, b, preferred_element_type=jnp.float32)


def _gat_stack_kernel(xT_ref, erow_ref, ecol_ref, W_ref, As_ref, Ad_ref,
                      b_ref, out_ref):
    srow = erow_ref[0:1, :]
    drow = erow_ref[1:2, :]
    vrow = erow_ref[2:3, :]
    dcol = ecol_ref[:, 0:1]

    n_iota = lax.broadcasted_iota(jnp.int32, (N, EP), 0).astype(jnp.float32)
    s_src = jnp.where(n_iota == srow, 1.0, 0.0)
    s_dstg = jnp.where(n_iota == drow, 1.0, 0.0)
    e_iota = lax.broadcasted_iota(jnp.int32, (EP, N), 1).astype(jnp.float32)
    s_dstT = jnp.where(e_iota == dcol, 1.0, 0.0)

    r_iota = lax.broadcasted_iota(jnp.int32, (HID, HP), 0)
    h_iota = lax.broadcasted_iota(jnp.int32, (HID, HP), 1)
    rexp = jnp.where(r_iota // D == h_iota, 1.0, 0.0)

    xT = xT_ref[...]
    for li in range(L):
        W = W_ref[li]
        a_s_mat = As_ref[li]
        a_d_mat = Ad_ref[li]
        b = b_ref[li]

        hT = _dot(W, xT)
        al_s = _dot(a_s_mat, hT)
        al_d = _dot(a_d_mat, hT)
        self_a = al_s + al_d
        self_a = jnp.where(self_a >= 0, self_a, 0.2 * self_a)

        ase = _dot(al_s, s_src)
        ade = _dot(al_d, s_dstg)
        ae = ase + ade
        ae = jnp.where(ae >= 0, ae, 0.2 * ae)
        ce = _dot(self_a, s_dstg)
        ee = jnp.exp(ae - ce) * vrow

        s_nodes = _dot(ee, s_dstT) + (1.0 + 1e-16)
        inv_s = 1.0 / s_nodes
        se = _dot(s_nodes, s_dstg)
        we = ee / (se + (1.0 - vrow))

        hs_e = _dot(hT, s_src)
        wexp = _dot(rexp, we)
        outT = _dot(hs_e * wexp, s_dstT)
        outT = outT + _dot(rexp, inv_s) * hT + b
        if li < L - 1:
            outT = jnp.where(outT > 0, outT,
                             jnp.exp(jnp.minimum(outT, 0.0)) - 1.0)
        xT = outT
    out_ref[...] = xT


def _gat_stack(xT, erow, ecol, Ws, As, Ad, bs):
    return pl.pallas_call(
        _gat_stack_kernel,
        out_shape=jax.ShapeDtypeStruct((HID, N), jnp.float32),
    )(xT, erow, ecol, Ws, As, Ad, bs)


def kernel(L_x_, L_edge_index_, L_self_modules_convs_modules_0_modules_lin_parameters_weight_, L_self_modules_convs_modules_0_parameters_att_src_, L_self_modules_convs_modules_0_parameters_att_dst_, L_self_modules_convs_modules_0_parameters_bias_, L_self_modules_convs_modules_1_modules_lin_parameters_weight_, L_self_modules_convs_modules_1_parameters_att_src_, L_self_modules_convs_modules_1_parameters_att_dst_, L_self_modules_convs_modules_1_parameters_bias_, L_self_modules_convs_modules_2_modules_lin_parameters_weight_, L_self_modules_convs_modules_2_parameters_att_src_, L_self_modules_convs_modules_2_parameters_att_dst_, L_self_modules_convs_modules_2_parameters_bias_, L_self_modules_convs_modules_3_modules_lin_parameters_weight_, L_self_modules_convs_modules_3_parameters_att_src_, L_self_modules_convs_modules_3_parameters_att_dst_, L_self_modules_convs_modules_3_parameters_bias_, L_self_modules_convs_modules_4_modules_lin_parameters_weight_, L_self_modules_convs_modules_4_parameters_att_src_, L_self_modules_convs_modules_4_parameters_att_dst_, L_self_modules_convs_modules_4_parameters_bias_):
    kw = dict(locals())
    x = kw['L_x_']
    ei = kw['L_edge_index_']
    src = ei[0].astype(jnp.float32)
    dst = ei[1].astype(jnp.float32)
    valid = (ei[0] != ei[1]).astype(jnp.float32)
    pad = EP - E
    srow = jnp.pad(src, (0, pad), constant_values=-1.0)
    drow = jnp.pad(dst, (0, pad), constant_values=-1.0)
    vrow = jnp.pad(valid, (0, pad))
    erow = jnp.zeros((8, EP), jnp.float32)
    erow = erow.at[0].set(srow).at[1].set(drow).at[2].set(vrow)
    ecol = jnp.zeros((EP, 8), jnp.float32).at[:, 0].set(
        jnp.where(vrow > 0, drow, -1.0))

    Ws, As, Ad, bs = [], [], [], []
    for li in range(L):
        W = kw['L_self_modules_convs_modules_%d_modules_lin_parameters_weight_' % li]
        a_s = kw['L_self_modules_convs_modules_%d_parameters_att_src_' % li]
        a_d = kw['L_self_modules_convs_modules_%d_parameters_att_dst_' % li]
        b = kw['L_self_modules_convs_modules_%d_parameters_bias_' % li]
        blk_s = jnp.zeros((HP, H, D), jnp.float32).at[:H].set(
            jnp.eye(H)[:, :, None] * a_s[0][None])
        blk_d = jnp.zeros((HP, H, D), jnp.float32).at[:H].set(
            jnp.eye(H)[:, :, None] * a_d[0][None])
        Ws.append(W)
        As.append(blk_s.reshape(HP, HID))
        Ad.append(blk_d.reshape(HP, HID))
        bs.append(b.reshape(HID, 1))

    outT = _gat_stack(x.T, erow, ecol,
                      jnp.stack(Ws), jnp.stack(As), jnp.stack(Ad),
                      jnp.stack(bs))
    return outT.T
